# Initial kernel scaffold; baseline (speedup 1.0000x reference)
import jax
import jax.numpy as jnp
from jax.experimental import pallas as pl


def kernel(x, W_g, b_g, W_e, b_e):
    raise NotImplementedError("write your pallas kernel here")


# speedup vs baseline: 1.9339x; 1.9339x over previous
import functools

import jax
import jax.numpy as jnp
from jax import lax
from jax.experimental import pallas as pl
from jax.experimental.pallas import tpu as pltpu
from jax.experimental.pallas import tpu_sc as plsc

HIDDEN = 2048
EXPERTS = 8
NC = 2
NS = 16
LANES = 16
NW = NC * NS


def _sc_sort_scatter(T):
    CH = T // NW
    NG = CH // LANES
    NGT = T // LANES
    mesh = plsc.VectorSubcoreMesh(core_axis_name="c", subcore_axis_name="s")

    @functools.partial(
        pl.kernel,
        mesh=mesh,
        compiler_params=pltpu.CompilerParams(needs_layout_passes=False),
        out_type=jax.ShapeDtypeStruct((T, HIDDEN), jnp.float32),
        scratch_types=[
            pltpu.VMEM((T,), jnp.int32),
            pltpu.VMEM((LANES, HIDDEN), jnp.float32),
            pltpu.SemaphoreType.DMA,
        ],
    )
    def body(e_hbm, x_hbm, xg_hbm, e_v, rows_v, sem):
        wid = lax.axis_index("s") * NC + lax.axis_index("c")
        base = wid * CH
        pltpu.sync_copy(e_hbm, e_v)
        lane = lax.iota(jnp.int32, LANES)

        def hist_body(i, carry):
            tot, bef = carry
            v = e_v[pl.ds(i * LANES, LANES)]
            add = jnp.zeros((LANES,), jnp.int32)
            for ex in range(EXPERTS):
                cnt = jnp.sum((v == ex).astype(jnp.int32))
                add = jnp.where(lane == ex, add + cnt, add)
            tot = tot + add
            bef = jnp.where(i * LANES < base, bef + add, bef)
            return tot, bef

        zero = jnp.zeros((LANES,), jnp.int32)
        tot, bef = lax.fori_loop(0, NGT, hist_body, (zero, zero))
        start = (plsc.cumsum(tot) - tot) + bef

        for j in range(NG):
            v = e_v[pl.ds(base + j * LANES, LANES)]
            pos = jnp.zeros((LANES,), jnp.int32)
            for ex in range(EXPERTS):
                m = v == ex
                mi = m.astype(jnp.int32)
                c = plsc.cumsum(mi)
                s_ex = jnp.sum(jnp.where(lane == ex, start, 0))
                pos = jnp.where(m, c - 1 + s_ex, pos)
                start = start + jnp.where(lane == ex, jnp.sum(mi), 0)
            pltpu.sync_copy(x_hbm.at[pl.ds(base + j * LANES, LANES)], rows_v)
            pltpu.async_copy(rows_v, xg_hbm.at[pos], sem).wait()

    return body


def _ffn_body(xg_ref, w_ref, b_ref, p_ref, o_ref):
    acc = jnp.dot(xg_ref[...], w_ref[...], preferred_element_type=jnp.float32)
    o_ref[...] = (acc + b_ref[...]) * p_ref[...]


def _tc_ffn(T, bt=512):
    return pl.pallas_call(
        _ffn_body,
        grid=(T // bt,),
        in_specs=[
            pl.BlockSpec((bt, HIDDEN), lambda i: (i, 0)),
            pl.BlockSpec((HIDDEN, HIDDEN), lambda i: (0, 0)),
            pl.BlockSpec((1, HIDDEN), lambda i: (0, 0)),
            pl.BlockSpec((bt, 1), lambda i: (i, 0)),
        ],
        out_specs=pl.BlockSpec((bt, HIDDEN), lambda i: (i, 0)),
        out_shape=jax.ShapeDtypeStruct((T, HIDDEN), jnp.float32),
    )


def kernel(x, W_g, b_g, W_e, b_e):
    B, S, D = x.shape
    T = B * S
    h = x.reshape(T, D)
    router_logits = h @ W_g + b_g
    normalized = jax.nn.softmax(router_logits, axis=1)
    e_idx = jnp.argmax(normalized, axis=1)
    prob = jnp.take_along_axis(normalized, e_idx[:, None], axis=1)[:, 0]
    xg = _sc_sort_scatter(T)(e_idx.astype(jnp.int32), h)
    out = _tc_ffn(T)(xg, W_e, b_e.reshape(1, D), prob.reshape(T, 1))
    return out.reshape(B, S, D)


# speedup vs baseline: 2.0180x; 1.0435x over previous
import functools

import jax
import jax.numpy as jnp
from jax import lax
from jax.experimental import pallas as pl
from jax.experimental.pallas import tpu as pltpu
from jax.experimental.pallas import tpu_sc as plsc

HIDDEN = 2048
EXPERTS = 8
NC = 2
NS = 16
LANES = 16
NW = NC * NS


def _sc_sort_scatter(T):
    CH = T // NW
    NG = CH // LANES
    NGT = T // LANES
    mesh = plsc.VectorSubcoreMesh(core_axis_name="c", subcore_axis_name="s")

    @functools.partial(
        pl.kernel,
        mesh=mesh,
        compiler_params=pltpu.CompilerParams(needs_layout_passes=False),
        out_type=jax.ShapeDtypeStruct((T, HIDDEN), jnp.float32),
        scratch_types=[
            pltpu.VMEM((T,), jnp.int32),
            pltpu.VMEM((2, LANES, HIDDEN), jnp.float32),
            pltpu.VMEM((LANES, LANES), jnp.int32),
            pltpu.SemaphoreType.DMA,
            pltpu.SemaphoreType.DMA,
        ],
    )
    def body(e_hbm, x_hbm, xg_hbm, e_v, rows_v, hist_v, sem_ld, sem_st):
        wid = lax.axis_index("s") * NC + lax.axis_index("c")
        base = wid * CH
        my_g0 = wid * NG
        pltpu.sync_copy(e_hbm, e_v)
        lane = lax.iota(jnp.int32, LANES)
        zero = jnp.zeros((LANES,), jnp.int32)
        ones = jnp.ones((LANES,), jnp.int32)
        lane16 = lane

        for l in range(LANES):
            hist_v[l] = zero

        def hist_body(i, _):
            v = e_v[pl.ds(i * LANES, LANES)]
            plsc.addupdate_scatter(hist_v, [lane16, v], ones)
            return 0

        def combine():
            acc = zero
            for l in range(LANES):
                acc = acc + hist_v[l]
            return acc

        lax.fori_loop(0, my_g0, hist_body, 0)
        bef = combine()
        lax.fori_loop(my_g0, NGT, hist_body, 0)
        tot = combine()
        start = (plsc.cumsum(tot) - tot) + bef

        pos_list = []
        for j in range(NG):
            v = e_v[pl.ds(base + j * LANES, LANES)]
            pos = zero
            for ex in range(EXPERTS):
                m = v == ex
                mi = m.astype(jnp.int32)
                c = plsc.cumsum(mi)
                s_ex = jnp.sum(jnp.where(lane == ex, start, 0))
                pos = jnp.where(m, c - 1 + s_ex, pos)
                start = start + jnp.where(lane == ex, jnp.sum(mi), 0)
            pos_list.append(pos)

        def load(j, b):
            return pltpu.async_copy(
                x_hbm.at[pl.ds(base + j * LANES, LANES)], rows_v.at[b], sem_ld)

        loads = {0: load(0, 0)}
        scats = {}
        for j in range(NG):
            if j + 1 < NG:
                if j - 1 >= 0:
                    scats[j - 1].wait()
                loads[j + 1] = load(j + 1, (j + 1) % 2)
            loads[j].wait()
            scats[j] = pltpu.async_copy(
                rows_v.at[j % 2], xg_hbm.at[pos_list[j]], sem_st)
        scats[NG - 2].wait()
        scats[NG - 1].wait()

    return body


def _ffn_body(xg_ref, w_ref, b_ref, p_ref, o_ref):
    acc = jnp.dot(xg_ref[...], w_ref[...], preferred_element_type=jnp.float32)
    o_ref[...] = (acc + b_ref[...]) * p_ref[...]


def _tc_ffn(T, bt=512):
    return pl.pallas_call(
        _ffn_body,
        grid=(T // bt,),
        in_specs=[
            pl.BlockSpec((bt, HIDDEN), lambda i: (i, 0)),
            pl.BlockSpec((HIDDEN, HIDDEN), lambda i: (0, 0)),
            pl.BlockSpec((1, HIDDEN), lambda i: (0, 0)),
            pl.BlockSpec((bt, 1), lambda i: (i, 0)),
        ],
        out_specs=pl.BlockSpec((bt, HIDDEN), lambda i: (i, 0)),
        out_shape=jax.ShapeDtypeStruct((T, HIDDEN), jnp.float32),
    )


def kernel(x, W_g, b_g, W_e, b_e):
    B, S, D = x.shape
    T = B * S
    h = x.reshape(T, D)
    router_logits = h @ W_g + b_g
    normalized = jax.nn.softmax(router_logits, axis=1)
    e_idx = jnp.argmax(normalized, axis=1)
    prob = jnp.take_along_axis(normalized, e_idx[:, None], axis=1)[:, 0]
    xg = _sc_sort_scatter(T)(e_idx.astype(jnp.int32), h)
    out = _tc_ffn(T)(xg, W_e, b_e.reshape(1, D), prob.reshape(T, 1))
    return out.reshape(B, S, D)
